# baseline (speedup 1.0000x reference)
import functools

import jax
import jax.numpy as jnp
from jax import lax
from jax.experimental import pallas as pl
from jax.experimental.pallas import tpu as pltpu
from jax.experimental.pallas import tpu_sc as plsc

_B = 16384
_H = 50
_V = 1000000
_D = 64
_HD = _D // 2
_TOT = _B * _H
_NC = 2
_NS = 16
_NW = _NC * _NS
_PER_W = _TOT // _NW
_CHUNK = 512
_IDXR = _CHUNK // 128
_NCHUNK = _PER_W // _CHUNK


_PACK_R = 8000
_PACK_R2 = _PACK_R // 2


def _pack_body(r_ref, i_ref, o_ref):
    rw = lax.bitcast_convert_type(r_ref[...].astype(jnp.float32), jnp.int32)
    iw = lax.bitcast_convert_type(i_ref[...].astype(jnp.float32), jnp.int32)

    def pk(w, lane0):
        lo = lax.shift_right_logical(w[:, lane0:lane0 + _HD], 16)
        hi = jnp.bitwise_and(w[:, lane0 + _HD:lane0 + _D], jnp.int32(-65536))
        return lo | hi

    z = jnp.zeros((_PACK_R2, _D), jnp.int32)
    even = jnp.concatenate([pk(rw, 0), pk(iw, 0), z], axis=1)
    odd = jnp.concatenate([pk(rw, _D), pk(iw, _D), z], axis=1)
    o_ref[...] = jnp.concatenate(
        [even[:, None, :], odd[:, None, :]], axis=1).reshape(_PACK_R, 128)


def _pack_tables(rt2, it2):
    return pl.pallas_call(
        _pack_body,
        grid=(_V // _PACK_R,),
        in_specs=[
            pl.BlockSpec((_PACK_R2, 128), lambda g: (g, 0)),
            pl.BlockSpec((_PACK_R2, 128), lambda g: (g, 0)),
        ],
        out_specs=pl.BlockSpec((_PACK_R, 128), lambda g: (g, 0)),
        out_shape=jax.ShapeDtypeStruct((_V, 128), jnp.int32),
    )(rt2, it2)


_mesh = plsc.VectorSubcoreMesh(core_axis_name="c", subcore_axis_name="s")


@functools.partial(
    pl.kernel,
    mesh=_mesh,
    out_type=jax.ShapeDtypeStruct((_TOT, 128), jnp.int32),
    scratch_types=[
        pltpu.VMEM((_IDXR, 128), jnp.int32),
        pltpu.VMEM((_CHUNK, 128), jnp.int32),
        pltpu.SemaphoreType.DMA,
    ],
)
def _sc_gather(x_hbm, tbl_hbm, out_hbm, idx_v, rows_v, sem):
    wid = lax.axis_index("s") * _NC + lax.axis_index("c")

    def chunk_body(g, carry):
        rowb = pl.multiple_of((wid * _NCHUNK + g) * _IDXR, _IDXR)
        base = pl.multiple_of((wid * _NCHUNK + g) * _CHUNK, _CHUNK)
        pltpu.sync_copy(x_hbm.at[pl.ds(rowb, _IDXR)], idx_v)
        cps = [
            pltpu.async_copy(tbl_hbm.at[idx_v.at[j]],
                             rows_v.at[pl.ds(j * 128, 128)], sem)
            for j in range(_IDXR)
        ]
        for cp in cps:
            cp.wait()
        pltpu.sync_copy(rows_v, out_hbm.at[pl.ds(base, _CHUNK)])
        return carry

    lax.fori_loop(0, _NCHUNK, chunk_body, 0)


_SPLIT_CB = 64
_SPLIT_R = _SPLIT_CB * _H


def _split_body(w_ref, r_ref, i_ref):
    w = w_ref[...]
    lo = lax.bitcast_convert_type(w << 16, jnp.float32).astype(jnp.bfloat16)
    hi = lax.bitcast_convert_type(
        jnp.bitwise_and(w, jnp.int32(-65536)), jnp.float32
    ).astype(jnp.bfloat16)
    r = jnp.concatenate([lo[:, :_HD], hi[:, :_HD]], axis=1)
    im = jnp.concatenate([lo[:, _HD:_D], hi[:, _HD:_D]], axis=1)
    r_ref[...] = r.reshape(_SPLIT_CB, _H, _D)
    i_ref[...] = im.reshape(_SPLIT_CB, _H, _D)


def _split(fused):
    return pl.pallas_call(
        _split_body,
        grid=(_B // _SPLIT_CB,),
        in_specs=[pl.BlockSpec((_SPLIT_R, 128), lambda g: (g, 0))],
        out_specs=[
            pl.BlockSpec((_SPLIT_CB, _H, _D), lambda g: (g, 0, 0)),
            pl.BlockSpec((_SPLIT_CB, _H, _D), lambda g: (g, 0, 0)),
        ],
        out_shape=[
            jax.ShapeDtypeStruct((_B, _H, _D), jnp.bfloat16),
            jax.ShapeDtypeStruct((_B, _H, _D), jnp.bfloat16),
        ],
    )(fused)


def kernel(x, real_table, imag_table):
    xf = x.reshape(_TOT // 128, 128)
    tbl = _pack_tables(
        real_table.astype(jnp.bfloat16).reshape(_V // 2, 128),
        imag_table.astype(jnp.bfloat16).reshape(_V // 2, 128))
    fused = _sc_gather(xf, tbl)
    real, imag = _split(fused)
    return (real, imag)


# speedup vs baseline: 1.0690x; 1.0690x over previous
import functools

import jax
import jax.numpy as jnp
from jax import lax
from jax.experimental import pallas as pl
from jax.experimental.pallas import tpu as pltpu
from jax.experimental.pallas import tpu_sc as plsc

_B = 16384
_H = 50
_V = 1000000
_D = 64
_HD = _D // 2
_TOT = _B * _H
_NC = 2
_NS = 16
_NW = _NC * _NS
_PER_W = _TOT // _NW
_CHUNK = 512
_IDXR = _CHUNK // 128
_NCHUNK = _PER_W // _CHUNK


_PACK_R = 8000
_PACK_R2 = _PACK_R // 2


def _pack_body(r_ref, i_ref, o_ref):
    rw = lax.bitcast_convert_type(
        r_ref[...].astype(jnp.bfloat16).astype(jnp.float32), jnp.int32)
    iw = lax.bitcast_convert_type(
        i_ref[...].astype(jnp.bfloat16).astype(jnp.float32), jnp.int32)

    def pk(w, lane0):
        lo = lax.shift_right_logical(w[:, lane0:lane0 + _HD], 16)
        hi = jnp.bitwise_and(w[:, lane0 + _HD:lane0 + _D], jnp.int32(-65536))
        return lo | hi

    z = jnp.zeros((_PACK_R2, _D), jnp.int32)
    even = jnp.concatenate([pk(rw, 0), pk(iw, 0), z], axis=1)
    odd = jnp.concatenate([pk(rw, _D), pk(iw, _D), z], axis=1)
    o_ref[...] = jnp.concatenate([even[None, :, :], odd[None, :, :]], axis=0)


def _pack_tables(rt2, it2):
    return pl.pallas_call(
        _pack_body,
        grid=(_V // _PACK_R,),
        in_specs=[
            pl.BlockSpec((_PACK_R2, 128), lambda g: (g, 0)),
            pl.BlockSpec((_PACK_R2, 128), lambda g: (g, 0)),
        ],
        out_specs=pl.BlockSpec((2, _PACK_R2, 128), lambda g: (0, g, 0)),
        out_shape=jax.ShapeDtypeStruct((2, _V // 2, 128), jnp.int32),
    )(rt2, it2)


_mesh = plsc.VectorSubcoreMesh(core_axis_name="c", subcore_axis_name="s")


@functools.partial(
    pl.kernel,
    mesh=_mesh,
    out_type=jax.ShapeDtypeStruct((_TOT, 128), jnp.int32),
    scratch_types=[
        pltpu.VMEM((_IDXR, 128), jnp.int32),
        pltpu.VMEM((_CHUNK, 128), jnp.int32),
        pltpu.SemaphoreType.DMA,
    ],
)
def _sc_gather(x_hbm, tbl_hbm, out_hbm, idx_v, rows_v, sem):
    wid = lax.axis_index("s") * _NC + lax.axis_index("c")

    def chunk_body(g, carry):
        rowb = pl.multiple_of((wid * _NCHUNK + g) * _IDXR, _IDXR)
        base = pl.multiple_of((wid * _NCHUNK + g) * _CHUNK, _CHUNK)
        pltpu.sync_copy(x_hbm.at[pl.ds(rowb, _IDXR)], idx_v)
        cps = [
            pltpu.async_copy(tbl_hbm.at[idx_v.at[j]],
                             rows_v.at[pl.ds(j * 128, 128)], sem)
            for j in range(_IDXR)
        ]
        for cp in cps:
            cp.wait()
        pltpu.sync_copy(rows_v, out_hbm.at[pl.ds(base, _CHUNK)])
        return carry

    lax.fori_loop(0, _NCHUNK, chunk_body, 0)


_SPLIT_CB = 64
_SPLIT_R = _SPLIT_CB * _H


def _split_body(w_ref, r_ref, i_ref):
    w = w_ref[...]
    lo = lax.bitcast_convert_type(w << 16, jnp.float32).astype(jnp.bfloat16)
    hi = lax.bitcast_convert_type(
        jnp.bitwise_and(w, jnp.int32(-65536)), jnp.float32
    ).astype(jnp.bfloat16)
    r = jnp.concatenate([lo[:, :_HD], hi[:, :_HD]], axis=1)
    im = jnp.concatenate([lo[:, _HD:_D], hi[:, _HD:_D]], axis=1)
    r_ref[...] = r.reshape(_SPLIT_CB, _H, _D)
    i_ref[...] = im.reshape(_SPLIT_CB, _H, _D)


def _split(fused):
    return pl.pallas_call(
        _split_body,
        grid=(_B // _SPLIT_CB,),
        in_specs=[pl.BlockSpec((_SPLIT_R, 128), lambda g: (g, 0))],
        out_specs=[
            pl.BlockSpec((_SPLIT_CB, _H, _D), lambda g: (g, 0, 0)),
            pl.BlockSpec((_SPLIT_CB, _H, _D), lambda g: (g, 0, 0)),
        ],
        out_shape=[
            jax.ShapeDtypeStruct((_B, _H, _D), jnp.bfloat16),
            jax.ShapeDtypeStruct((_B, _H, _D), jnp.bfloat16),
        ],
    )(fused)


def kernel(x, real_table, imag_table):
    xr = x.reshape(_TOT // 128, 128)
    xf = (x.reshape(_TOT // 128, 128) & 1) * (_V // 2) + (xr >> 1)
    tbl = _pack_tables(real_table.reshape(_V // 2, 128),
                       imag_table.reshape(_V // 2, 128))
    fused = _sc_gather(xf, tbl.reshape(_V, 128))
    real, imag = _split(fused)
    return (real, imag)


# speedup vs baseline: 1.2315x; 1.1521x over previous
import functools

import jax
import jax.numpy as jnp
from jax import lax
from jax.experimental import pallas as pl
from jax.experimental.pallas import tpu as pltpu
from jax.experimental.pallas import tpu_sc as plsc

_B = 16384
_H = 50
_V = 1000000
_D = 64
_HD = _D // 2
_TOT = _B * _H
_NC = 2
_NS = 16
_NW = _NC * _NS
_PER_W = _TOT // _NW
_CHUNK = 512
_IDXR = _CHUNK // 128
_NCHUNK = _PER_W // _CHUNK

_PACK_R = 8000


def _pack_body(r_ref, i_ref, o_ref):
    def pk(b):
        w = lax.bitcast_convert_type(b.astype(jnp.float32), jnp.int32)
        lo = lax.shift_right_logical(w[:, :_HD], 16)
        hi = jnp.bitwise_and(w[:, _HD:], jnp.int32(-65536))
        return lo | hi

    o_ref[...] = jnp.concatenate(
        [pk(r_ref[...]), pk(i_ref[...]),
         jnp.zeros((_PACK_R, _D), jnp.int32)], axis=1)


def _pack_tables(rt, it):
    return pl.pallas_call(
        _pack_body,
        grid=(_V // _PACK_R,),
        in_specs=[
            pl.BlockSpec((_PACK_R, _D), lambda g: (g, 0)),
            pl.BlockSpec((_PACK_R, _D), lambda g: (g, 0)),
        ],
        out_specs=pl.BlockSpec((_PACK_R, 128), lambda g: (g, 0)),
        out_shape=jax.ShapeDtypeStruct((_V, 128), jnp.int32),
    )(rt, it)


_mesh = plsc.VectorSubcoreMesh(core_axis_name="c", subcore_axis_name="s")


@functools.partial(
    pl.kernel,
    mesh=_mesh,
    out_type=jax.ShapeDtypeStruct((_TOT, 128), jnp.int32),
    scratch_types=[
        pltpu.VMEM((_IDXR, 128), jnp.int32),
        pltpu.VMEM((_CHUNK, 128), jnp.int32),
        pltpu.SemaphoreType.DMA,
    ],
)
def _sc_gather(x_hbm, tbl_hbm, out_hbm, idx_v, rows_v, sem):
    wid = lax.axis_index("s") * _NC + lax.axis_index("c")

    def chunk_body(g, carry):
        rowb = pl.multiple_of((wid * _NCHUNK + g) * _IDXR, _IDXR)
        base = pl.multiple_of((wid * _NCHUNK + g) * _CHUNK, _CHUNK)
        pltpu.sync_copy(x_hbm.at[pl.ds(rowb, _IDXR)], idx_v)
        cps = [
            pltpu.async_copy(tbl_hbm.at[idx_v.at[j]],
                             rows_v.at[pl.ds(j * 128, 128)], sem)
            for j in range(_IDXR)
        ]
        for cp in cps:
            cp.wait()
        pltpu.sync_copy(rows_v, out_hbm.at[pl.ds(base, _CHUNK)])
        return carry

    lax.fori_loop(0, _NCHUNK, chunk_body, 0)


_SPLIT_CB = 64
_SPLIT_R = _SPLIT_CB * _H


def _split_body(w_ref, r_ref, i_ref):
    w = w_ref[...]
    lo = lax.bitcast_convert_type(w << 16, jnp.float32).astype(jnp.bfloat16)
    hi = lax.bitcast_convert_type(
        jnp.bitwise_and(w, jnp.int32(-65536)), jnp.float32
    ).astype(jnp.bfloat16)
    r = jnp.concatenate([lo[:, :_HD], hi[:, :_HD]], axis=1)
    im = jnp.concatenate([lo[:, _HD:_D], hi[:, _HD:_D]], axis=1)
    r_ref[...] = r.reshape(_SPLIT_CB, _H, _D)
    i_ref[...] = im.reshape(_SPLIT_CB, _H, _D)


def _split(fused):
    return pl.pallas_call(
        _split_body,
        grid=(_B // _SPLIT_CB,),
        in_specs=[pl.BlockSpec((_SPLIT_R, 128), lambda g: (g, 0))],
        out_specs=[
            pl.BlockSpec((_SPLIT_CB, _H, _D), lambda g: (g, 0, 0)),
            pl.BlockSpec((_SPLIT_CB, _H, _D), lambda g: (g, 0, 0)),
        ],
        out_shape=[
            jax.ShapeDtypeStruct((_B, _H, _D), jnp.bfloat16),
            jax.ShapeDtypeStruct((_B, _H, _D), jnp.bfloat16),
        ],
    )(fused)


def kernel(x, real_table, imag_table):
    xf = x.reshape(_TOT // 128, 128)
    tbl = _pack_tables(real_table.astype(jnp.bfloat16),
                       imag_table.astype(jnp.bfloat16))
    fused = _sc_gather(xf, tbl)
    real, imag = _split(fused)
    return (real, imag)


# speedup vs baseline: 1.2814x; 1.0405x over previous
import functools

import jax
import jax.numpy as jnp
from jax import lax
from jax.experimental import pallas as pl
from jax.experimental.pallas import tpu as pltpu
from jax.experimental.pallas import tpu_sc as plsc

_B = 16384
_H = 50
_V = 1000000
_D = 64
_HD = _D // 2
_TOT = _B * _H
_NC = 2
_NS = 16
_NW = _NC * _NS
_PER_W = _TOT // _NW
_CHUNK = 512
_IDXR = _CHUNK // 128
_NCHUNK = _PER_W // _CHUNK

_PACK_R = 20000


def _pack_body(r_ref, i_ref, o_ref):
    def pk(b):
        w = lax.bitcast_convert_type(b.astype(jnp.float32), jnp.int32)
        lo = lax.shift_right_logical(w[:, :_HD], 16)
        hi = jnp.bitwise_and(w[:, _HD:], jnp.int32(-65536))
        return lo | hi

    o_ref[...] = jnp.concatenate(
        [pk(r_ref[...]), pk(i_ref[...]),
         jnp.zeros((_PACK_R, _D), jnp.int32)], axis=1)


def _pack_tables(rt, it):
    return pl.pallas_call(
        _pack_body,
        grid=(_V // _PACK_R,),
        in_specs=[
            pl.BlockSpec((_PACK_R, _D), lambda g: (g, 0)),
            pl.BlockSpec((_PACK_R, _D), lambda g: (g, 0)),
        ],
        out_specs=pl.BlockSpec((_PACK_R, 128), lambda g: (g, 0)),
        out_shape=jax.ShapeDtypeStruct((_V, 128), jnp.int32),
    )(rt, it)


_mesh = plsc.VectorSubcoreMesh(core_axis_name="c", subcore_axis_name="s")


@functools.partial(
    pl.kernel,
    mesh=_mesh,
    out_type=jax.ShapeDtypeStruct((_TOT, 128), jnp.int32),
    scratch_types=[
        pltpu.VMEM((_IDXR, 128), jnp.int32),
        pltpu.VMEM((_CHUNK, 128), jnp.int32),
        pltpu.SemaphoreType.DMA,
    ],
)
def _sc_gather(x_hbm, tbl_hbm, out_hbm, idx_v, rows_v, sem):
    wid = lax.axis_index("s") * _NC + lax.axis_index("c")

    def chunk_body(g, carry):
        rowb = pl.multiple_of((wid * _NCHUNK + g) * _IDXR, _IDXR)
        base = pl.multiple_of((wid * _NCHUNK + g) * _CHUNK, _CHUNK)
        pltpu.sync_copy(x_hbm.at[pl.ds(rowb, _IDXR)], idx_v)
        cps = [
            pltpu.async_copy(tbl_hbm.at[idx_v.at[j]],
                             rows_v.at[pl.ds(j * 128, 128)], sem)
            for j in range(_IDXR)
        ]
        for cp in cps:
            cp.wait()
        pltpu.sync_copy(rows_v, out_hbm.at[pl.ds(base, _CHUNK)])
        return carry

    lax.fori_loop(0, _NCHUNK, chunk_body, 0)


_SPLIT_CB = 128
_SPLIT_R = _SPLIT_CB * _H


def _split_body(w_ref, r_ref, i_ref):
    w = w_ref[...]
    lo = lax.bitcast_convert_type(w << 16, jnp.float32).astype(jnp.bfloat16)
    hi = lax.bitcast_convert_type(
        jnp.bitwise_and(w, jnp.int32(-65536)), jnp.float32
    ).astype(jnp.bfloat16)
    r = jnp.concatenate([lo[:, :_HD], hi[:, :_HD]], axis=1)
    im = jnp.concatenate([lo[:, _HD:_D], hi[:, _HD:_D]], axis=1)
    r_ref[...] = r.reshape(_SPLIT_CB, _H, _D)
    i_ref[...] = im.reshape(_SPLIT_CB, _H, _D)


def _split(fused):
    return pl.pallas_call(
        _split_body,
        grid=(_B // _SPLIT_CB,),
        in_specs=[pl.BlockSpec((_SPLIT_R, 128), lambda g: (g, 0))],
        out_specs=[
            pl.BlockSpec((_SPLIT_CB, _H, _D), lambda g: (g, 0, 0)),
            pl.BlockSpec((_SPLIT_CB, _H, _D), lambda g: (g, 0, 0)),
        ],
        out_shape=[
            jax.ShapeDtypeStruct((_B, _H, _D), jnp.bfloat16),
            jax.ShapeDtypeStruct((_B, _H, _D), jnp.bfloat16),
        ],
    )(fused)


def kernel(x, real_table, imag_table):
    xf = x.reshape(_TOT // 128, 128)
    tbl = _pack_tables(real_table.astype(jnp.bfloat16),
                       imag_table.astype(jnp.bfloat16))
    fused = _sc_gather(xf, tbl)
    real, imag = _split(fused)
    return (real, imag)
